# baseline (speedup 1.0000x reference)
import functools

import jax
import jax.numpy as jnp
from jax import lax
from jax.experimental import pallas as pl
from jax.experimental.pallas import tpu as pltpu
from jax.experimental.pallas import tpu_sc as plsc

N = 100000
D = 384
K = 8
H = 128
E = 6400000
WALK_STEPS = 3

BM = 2048
ROWS_PER_TILE = 6272
N_ACC = 16 * ROWS_PER_TILE
GRID_M = N_ACC // BM

NWORK = 32
CHUNK = 128
MROWS = 8
MACRO = MROWS * CHUNK
MPW = 196
E_PAD = NWORK * MPW * MACRO


def _mlp_body(q, emb, labf, wp1a, wp1b, bp1, wp2, bp2, wc1a, wc1b, bc1, wc2,
              bc2, amps_o, y_o):
    e = emb[...]
    qv = q[...]
    bp = bp1[...] + jnp.dot(qv, wp1b[...], preferred_element_type=jnp.float32)
    hp = jnp.maximum(jnp.dot(e, wp1a[...], preferred_element_type=jnp.float32) + bp, 0.0)
    ps = jax.nn.sigmoid(jnp.dot(hp, wp2[...], preferred_element_type=jnp.float32) + bp2[...])
    ps = ps + labf[...]
    bc = bc1[...] + jnp.dot(qv, wc1b[...], preferred_element_type=jnp.float32)
    hc = jnp.maximum(jnp.dot(e, wc1a[...], preferred_element_type=jnp.float32) + bc, 0.0)
    amps = (jnp.dot(hc, wc2[...], preferred_element_type=jnp.float32) + bc2[...]) * (1.0 + ps)
    rowi = lax.broadcasted_iota(jnp.int32, (BM, 1), 0) + pl.program_id(0) * BM
    amps = jnp.where(rowi < N, amps, 0.0)
    amps_o[...] = amps
    y_o[...] = amps * (1.0 / jnp.sqrt(jnp.float32(N * K)))


def _mlp(q2, emb, labf, wp1a, wp1b, bp1, wp2, bp2, wc1a, wc1b, bc1, wc2, bc2):
    rep = lambda i: (0, 0)
    return pl.pallas_call(
        _mlp_body,
        grid=(GRID_M,),
        in_specs=[
            pl.BlockSpec((1, D), rep),
            pl.BlockSpec((BM, D), lambda i: (i, 0)),
            pl.BlockSpec((BM, 1), lambda i: (i, 0)),
            pl.BlockSpec((D, H), rep),
            pl.BlockSpec((D, H), rep),
            pl.BlockSpec((1, H), rep),
            pl.BlockSpec((H, 1), rep),
            pl.BlockSpec((1, 1), rep),
            pl.BlockSpec((D, H), rep),
            pl.BlockSpec((D, H), rep),
            pl.BlockSpec((1, H), rep),
            pl.BlockSpec((H, K), rep),
            pl.BlockSpec((1, K), rep),
        ],
        out_specs=[
            pl.BlockSpec((BM, K), lambda i: (i, 0)),
            pl.BlockSpec((BM, K), lambda i: (i, 0)),
        ],
        out_shape=[
            jax.ShapeDtypeStruct((N_ACC, K), jnp.float32),
            jax.ShapeDtypeStruct((N_ACC, K), jnp.float32),
        ],
    )(q2, emb, labf, wp1a, wp1b, bp1, wp2, bp2, wc1a, wc1b, bc1, wc2, bc2)


def _spmm_body(cols_hbm, rows_hbm, y_hbm, zeros_hbm, out_hbm,
               colv, rowv, gbuf, acc, yv, semA, semB):
    c = lax.axis_index("c")
    s = lax.axis_index("s")
    wid = c * 16 + s
    row0 = s * ROWS_PER_TILE
    pltpu.sync_copy(zeros_hbm, acc.at[pl.ds(row0, ROWS_PER_TILE)])
    pltpu.sync_copy(y_hbm.at[pl.ds(row0, ROWS_PER_TILE)],
                    yv.at[pl.ds(row0, ROWS_PER_TILE)])
    plsc.subcore_barrier()

    def stage(m, b):
        mbase = (wid * MPW + m) * MROWS
        pltpu.sync_copy(cols_hbm.at[pl.ds(mbase, MROWS)], colv.at[b])
        pltpu.sync_copy(rows_hbm.at[pl.ds(mbase, MROWS)], rowv.at[b])

    def fire(b, sem):
        for j in range(MROWS):
            pltpu.async_copy(yv.at[colv.at[b, j]], gbuf.at[b, j], sem)

    def drain(b, sem):
        for j in range(MROWS):
            pltpu.make_async_copy(yv.at[colv.at[b, j]], gbuf.at[b, j], sem).wait()

    def scatter(b):
        for j in range(MROWS):
            pltpu.sync_copy(gbuf.at[b, j], acc.at[rowv.at[b, j]], add=True)

    stage(0, 0)
    fire(0, semA)

    def body(i, carry):
        m = 2 * i
        stage(m + 1, 1)
        fire(1, semB)
        drain(0, semA)
        scatter(0)

        @pl.when(m + 2 < MPW)
        def _():
            stage(m + 2, 0)
            fire(0, semA)

        drain(1, semB)
        scatter(1)
        return carry

    lax.fori_loop(0, MPW // 2, body, 0)
    plsc.subcore_barrier()
    pltpu.sync_copy(acc.at[pl.ds(row0, ROWS_PER_TILE)],
                    out_hbm.at[c, pl.ds(row0, ROWS_PER_TILE)])


_spmm = functools.partial(
    pl.kernel,
    mesh=plsc.VectorSubcoreMesh(core_axis_name="c", subcore_axis_name="s"),
    compiler_params=pltpu.CompilerParams(use_tc_tiling_on_sc=False),
    out_type=jax.ShapeDtypeStruct((2, N_ACC, K), jnp.float32),
    scratch_types=[
        pltpu.VMEM((2, MROWS, CHUNK), jnp.int32),
        pltpu.VMEM((2, MROWS, CHUNK), jnp.int32),
        pltpu.VMEM((2, MROWS, CHUNK, K), jnp.float32),
        pltpu.VMEM_SHARED((N_ACC, K), jnp.float32),
        pltpu.VMEM_SHARED((N_ACC, K), jnp.float32),
        pltpu.SemaphoreType.DMA,
        pltpu.SemaphoreType.DMA,
    ],
)(_spmm_body)


def _combine_body(p, amps, y_o):
    pv = p[...]
    y_o[...] = (pv[0] + pv[1]) * amps[...]


def _combine(p, amps):
    return pl.pallas_call(
        _combine_body,
        grid=(GRID_M,),
        in_specs=[
            pl.BlockSpec((2, BM, K), lambda i: (0, i, 0)),
            pl.BlockSpec((BM, K), lambda i: (i, 0)),
        ],
        out_specs=pl.BlockSpec((BM, K), lambda i: (i, 0)),
        out_shape=jax.ShapeDtypeStruct((N_ACC, K), jnp.float32),
    )(p, amps)


def _final_body(p, ra_o, ss_o):
    pv = p[...]
    u = pv[0] + pv[1]
    rowi = lax.broadcasted_iota(jnp.int32, (BM, 1), 0) + pl.program_id(0) * BM
    u = jnp.where(rowi < N, u, 0.0)
    ra_o[...] = jnp.sum(jnp.abs(u), axis=1, keepdims=True)
    blk = jnp.sum(u * u)
    prev = jnp.where(pl.program_id(0) == 0, 0.0, ss_o[0, 0])
    ss_o[0, 0] = prev + blk


def _final(p):
    return pl.pallas_call(
        _final_body,
        grid=(GRID_M,),
        in_specs=[pl.BlockSpec((2, BM, K), lambda i: (0, i, 0))],
        out_specs=[
            pl.BlockSpec((BM, 1), lambda i: (i, 0)),
            pl.BlockSpec(memory_space=pltpu.SMEM),
        ],
        out_shape=[
            jax.ShapeDtypeStruct((N_ACC, 1), jnp.float32),
            jax.ShapeDtypeStruct((1, 1), jnp.float32),
        ],
    )(p)


def _scale_body(ra, ss, o):
    ssv = ss[0, 0]
    inv = jnp.where(ssv > 0.0, lax.rsqrt(ssv), 1.0)
    o[...] = ra[...] * inv


def _scale(ra, ss):
    return pl.pallas_call(
        _scale_body,
        grid=(GRID_M,),
        in_specs=[
            pl.BlockSpec((BM, 1), lambda i: (i, 0)),
            pl.BlockSpec(memory_space=pltpu.SMEM),
        ],
        out_specs=pl.BlockSpec((BM, 1), lambda i: (i, 0)),
        out_shape=jax.ShapeDtypeStruct((N_ACC, 1), jnp.float32),
    )(ra, ss)


def kernel(q_emb, emb, edge_index, labels, Wc1, bc1, Wc2, bc2, Wp1, bp1, Wp2, bp2):
    q2 = q_emb.reshape(1, D)
    labf = labels.astype(jnp.float32).reshape(N, 1)
    amps, y = _mlp(
        q2, emb, labf,
        Wp1[:D], Wp1[D:], bp1.reshape(1, H), Wp2, bp2.reshape(1, 1),
        Wc1[:D], Wc1[D:], bc1.reshape(1, H), Wc2, bc2.reshape(1, K),
    )
    npad = E_PAD - E
    cols = jnp.concatenate([edge_index[1], jnp.zeros((npad,), jnp.int32)])
    rows = jnp.concatenate([edge_index[0], jnp.full((npad,), N, jnp.int32)])
    cols2 = cols.reshape(E_PAD // CHUNK, CHUNK)
    rows2 = rows.reshape(E_PAD // CHUNK, CHUNK)
    zeros = jnp.zeros((ROWS_PER_TILE, K), jnp.float32)
    for step in range(WALK_STEPS):
        p = _spmm(cols2, rows2, y, zeros)
        if step < WALK_STEPS - 1:
            y = _combine(p, amps)
    ra, ss = _final(p)
    out = _scale(ra, ss)
    return out[:N].reshape(N)


# speedup vs baseline: 1.4326x; 1.4326x over previous
import functools

import jax
import jax.numpy as jnp
from jax import lax
from jax.experimental import pallas as pl
from jax.experimental.pallas import tpu as pltpu
from jax.experimental.pallas import tpu_sc as plsc

N = 100000
D = 384
K = 8
H = 128
E = 6400000
WALK_STEPS = 3

BM = 2048
ROWS_PER_TILE = 6272
N_ACC = 16 * ROWS_PER_TILE
GRID_M = N_ACC // BM

NWORK = 32
CHUNK = 128
MROWS = 4
MACRO = MROWS * CHUNK
MPW = 392
RING = 4
E_PAD = NWORK * MPW * MACRO


def _mlp_body(q, emb, labf, wp1a, wp1b, bp1, wp2, bp2, wc1a, wc1b, bc1, wc2,
              bc2, amps_o, y_o):
    e = emb[...]
    qv = q[...]
    bp = bp1[...] + jnp.dot(qv, wp1b[...], preferred_element_type=jnp.float32)
    hp = jnp.maximum(jnp.dot(e, wp1a[...], preferred_element_type=jnp.float32) + bp, 0.0)
    ps = jax.nn.sigmoid(jnp.dot(hp, wp2[...], preferred_element_type=jnp.float32) + bp2[...])
    ps = ps + labf[...]
    bc = bc1[...] + jnp.dot(qv, wc1b[...], preferred_element_type=jnp.float32)
    hc = jnp.maximum(jnp.dot(e, wc1a[...], preferred_element_type=jnp.float32) + bc, 0.0)
    amps = (jnp.dot(hc, wc2[...], preferred_element_type=jnp.float32) + bc2[...]) * (1.0 + ps)
    rowi = lax.broadcasted_iota(jnp.int32, (BM, 1), 0) + pl.program_id(0) * BM
    amps = jnp.where(rowi < N, amps, 0.0)
    amps_o[...] = amps
    y_o[...] = amps * (1.0 / jnp.sqrt(jnp.float32(N * K)))


def _mlp(q2, emb, labf, wp1a, wp1b, bp1, wp2, bp2, wc1a, wc1b, bc1, wc2, bc2):
    rep = lambda i: (0, 0)
    return pl.pallas_call(
        _mlp_body,
        grid=(GRID_M,),
        in_specs=[
            pl.BlockSpec((1, D), rep),
            pl.BlockSpec((BM, D), lambda i: (i, 0)),
            pl.BlockSpec((BM, 1), lambda i: (i, 0)),
            pl.BlockSpec((D, H), rep),
            pl.BlockSpec((D, H), rep),
            pl.BlockSpec((1, H), rep),
            pl.BlockSpec((H, 1), rep),
            pl.BlockSpec((1, 1), rep),
            pl.BlockSpec((D, H), rep),
            pl.BlockSpec((D, H), rep),
            pl.BlockSpec((1, H), rep),
            pl.BlockSpec((H, K), rep),
            pl.BlockSpec((1, K), rep),
        ],
        out_specs=[
            pl.BlockSpec((BM, K), lambda i: (i, 0)),
            pl.BlockSpec((BM, K), lambda i: (i, 0)),
        ],
        out_shape=[
            jax.ShapeDtypeStruct((N_ACC, K), jnp.float32),
            jax.ShapeDtypeStruct((N_ACC, K), jnp.float32),
        ],
    )(q2, emb, labf, wp1a, wp1b, bp1, wp2, bp2, wc1a, wc1b, bc1, wc2, bc2)


def _spmm_body(cols_hbm, rows_hbm, y_hbm, zeros_hbm, out_hbm,
               colv, rowv, gbuf, acc, yv,
               semI0, semI1, semI2, semI3,
               semG0, semG1, semG2, semG3,
               semS0, semS1, semS2, semS3):
    c = lax.axis_index("c")
    s = lax.axis_index("s")
    wid = c * 16 + s
    row0 = s * ROWS_PER_TILE
    semI = (semI0, semI1, semI2, semI3)
    semG = (semG0, semG1, semG2, semG3)
    semS = (semS0, semS1, semS2, semS3)
    pltpu.sync_copy(zeros_hbm, acc.at[pl.ds(row0, ROWS_PER_TILE)])
    pltpu.sync_copy(y_hbm.at[pl.ds(row0, ROWS_PER_TILE)],
                    yv.at[pl.ds(row0, ROWS_PER_TILE)])
    plsc.subcore_barrier()

    def stage_f(m, b):
        mbase = (wid * MPW + m) * MROWS
        pltpu.async_copy(cols_hbm.at[pl.ds(mbase, MROWS)], colv.at[b], semI[b])
        pltpu.async_copy(rows_hbm.at[pl.ds(mbase, MROWS)], rowv.at[b], semI[b])

    def stage_d(m, b):
        mbase = (wid * MPW + m) * MROWS
        pltpu.make_async_copy(cols_hbm.at[pl.ds(mbase, MROWS)], colv.at[b], semI[b]).wait()
        pltpu.make_async_copy(rows_hbm.at[pl.ds(mbase, MROWS)], rowv.at[b], semI[b]).wait()

    def fire_g(b):
        for j in range(MROWS):
            pltpu.async_copy(yv.at[colv.at[b, j]], gbuf.at[b, j], semG[b])

    def drain_g(b):
        for j in range(MROWS):
            pltpu.make_async_copy(yv.at[colv.at[b, j]], gbuf.at[b, j], semG[b]).wait()

    def fire_s(b):
        for j in range(MROWS):
            pltpu.async_copy(gbuf.at[b, j], acc.at[rowv.at[b, j]], semS[b], add=True)

    def drain_s(b):
        for j in range(MROWS):
            pltpu.make_async_copy(gbuf.at[b, j], acc.at[rowv.at[b, j]], semS[b]).wait()

    for b in range(RING):
        stage_f(b, b)

    def body(i, carry):
        m = i * RING
        for b in range(RING):
            stage_d(m + b, b)
            fire_g(b)
        for b in range(RING):
            drain_g(b)
            fire_s(b)
        for b in range(RING):
            @pl.when(m + RING + b < MPW)
            def _():
                drain_s(b)
                stage_f(m + RING + b, b)
        return carry

    lax.fori_loop(0, MPW // RING, body, 0)
    for b in range(RING):
        drain_s(b)
    plsc.subcore_barrier()
    pltpu.sync_copy(acc.at[pl.ds(row0, ROWS_PER_TILE)],
                    out_hbm.at[c, pl.ds(row0, ROWS_PER_TILE)])


_spmm = functools.partial(
    pl.kernel,
    mesh=plsc.VectorSubcoreMesh(core_axis_name="c", subcore_axis_name="s"),
    compiler_params=pltpu.CompilerParams(use_tc_tiling_on_sc=False),
    out_type=jax.ShapeDtypeStruct((2, N_ACC, K), jnp.float32),
    scratch_types=[
        pltpu.VMEM((RING, MROWS, CHUNK), jnp.int32),
        pltpu.VMEM((RING, MROWS, CHUNK), jnp.int32),
        pltpu.VMEM((RING, MROWS, CHUNK, K), jnp.float32),
        pltpu.VMEM_SHARED((N_ACC, K), jnp.float32),
        pltpu.VMEM_SHARED((N_ACC, K), jnp.float32),
    ] + [pltpu.SemaphoreType.DMA] * 12,
)(_spmm_body)


def _combine_body(p, amps, y_o):
    pv = p[...]
    y_o[...] = (pv[0] + pv[1]) * amps[...]


def _combine(p, amps):
    return pl.pallas_call(
        _combine_body,
        grid=(GRID_M,),
        in_specs=[
            pl.BlockSpec((2, BM, K), lambda i: (0, i, 0)),
            pl.BlockSpec((BM, K), lambda i: (i, 0)),
        ],
        out_specs=pl.BlockSpec((BM, K), lambda i: (i, 0)),
        out_shape=jax.ShapeDtypeStruct((N_ACC, K), jnp.float32),
    )(p, amps)


def _final_body(p, ra_o, ss_o):
    pv = p[...]
    u = pv[0] + pv[1]
    rowi = lax.broadcasted_iota(jnp.int32, (BM, 1), 0) + pl.program_id(0) * BM
    u = jnp.where(rowi < N, u, 0.0)
    ra_o[...] = jnp.sum(jnp.abs(u), axis=1, keepdims=True)
    blk = jnp.sum(u * u)
    prev = jnp.where(pl.program_id(0) == 0, 0.0, ss_o[0, 0])
    ss_o[0, 0] = prev + blk


def _final(p):
    return pl.pallas_call(
        _final_body,
        grid=(GRID_M,),
        in_specs=[pl.BlockSpec((2, BM, K), lambda i: (0, i, 0))],
        out_specs=[
            pl.BlockSpec((BM, 1), lambda i: (i, 0)),
            pl.BlockSpec(memory_space=pltpu.SMEM),
        ],
        out_shape=[
            jax.ShapeDtypeStruct((N_ACC, 1), jnp.float32),
            jax.ShapeDtypeStruct((1, 1), jnp.float32),
        ],
    )(p)


def _scale_body(ra, ss, o):
    ssv = ss[0, 0]
    inv = jnp.where(ssv > 0.0, lax.rsqrt(ssv), 1.0)
    o[...] = ra[...] * inv


def _scale(ra, ss):
    return pl.pallas_call(
        _scale_body,
        grid=(GRID_M,),
        in_specs=[
            pl.BlockSpec((BM, 1), lambda i: (i, 0)),
            pl.BlockSpec(memory_space=pltpu.SMEM),
        ],
        out_specs=pl.BlockSpec((BM, 1), lambda i: (i, 0)),
        out_shape=jax.ShapeDtypeStruct((N_ACC, 1), jnp.float32),
    )(ra, ss)


def kernel(q_emb, emb, edge_index, labels, Wc1, bc1, Wc2, bc2, Wp1, bp1, Wp2, bp2):
    q2 = q_emb.reshape(1, D)
    labf = labels.astype(jnp.float32).reshape(N, 1)
    amps, y = _mlp(
        q2, emb, labf,
        Wp1[:D], Wp1[D:], bp1.reshape(1, H), Wp2, bp2.reshape(1, 1),
        Wc1[:D], Wc1[D:], bc1.reshape(1, H), Wc2, bc2.reshape(1, K),
    )
    npad = E_PAD - E
    cols = jnp.concatenate([edge_index[1], jnp.zeros((npad,), jnp.int32)])
    rows = jnp.concatenate([edge_index[0], jnp.full((npad,), N, jnp.int32)])
    cols2 = cols.reshape(E_PAD // CHUNK, CHUNK)
    rows2 = rows.reshape(E_PAD // CHUNK, CHUNK)
    zeros = jnp.zeros((ROWS_PER_TILE, K), jnp.float32)
    for step in range(WALK_STEPS):
        p = _spmm(cols2, rows2, y, zeros)
        if step < WALK_STEPS - 1:
            y = _combine(p, amps)
    ra, ss = _final(p)
    out = _scale(ra, ss)
    return out[:N].reshape(N)


# speedup vs baseline: 1.4632x; 1.0214x over previous
import functools

import jax
import jax.numpy as jnp
from jax import lax
from jax.experimental import pallas as pl
from jax.experimental.pallas import tpu as pltpu
from jax.experimental.pallas import tpu_sc as plsc

N = 100000
D = 384
K = 8
H = 128
E = 6400000
WALK_STEPS = 3

BM = 2048
ROWS_PER_TILE = 6272
N_ACC = 16 * ROWS_PER_TILE
GRID_M = N_ACC // BM

NWORK = 32
CHUNK = 128
MROWS = 2
MACRO = MROWS * CHUNK
MPW = 784
RING = 8
E_PAD = NWORK * MPW * MACRO


def _mlp_body(q, emb, labf, wp1a, wp1b, bp1, wp2, bp2, wc1a, wc1b, bc1, wc2,
              bc2, amps_o, y_o):
    e = emb[...]
    qv = q[...]
    bp = bp1[...] + jnp.dot(qv, wp1b[...], preferred_element_type=jnp.float32)
    hp = jnp.maximum(jnp.dot(e, wp1a[...], preferred_element_type=jnp.float32) + bp, 0.0)
    ps = jax.nn.sigmoid(jnp.dot(hp, wp2[...], preferred_element_type=jnp.float32) + bp2[...])
    ps = ps + labf[...]
    bc = bc1[...] + jnp.dot(qv, wc1b[...], preferred_element_type=jnp.float32)
    hc = jnp.maximum(jnp.dot(e, wc1a[...], preferred_element_type=jnp.float32) + bc, 0.0)
    amps = (jnp.dot(hc, wc2[...], preferred_element_type=jnp.float32) + bc2[...]) * (1.0 + ps)
    rowi = lax.broadcasted_iota(jnp.int32, (BM, 1), 0) + pl.program_id(0) * BM
    amps = jnp.where(rowi < N, amps, 0.0)
    amps_o[...] = amps
    y_o[...] = amps * (1.0 / jnp.sqrt(jnp.float32(N * K)))


def _mlp(q2, emb, labf, wp1a, wp1b, bp1, wp2, bp2, wc1a, wc1b, bc1, wc2, bc2):
    rep = lambda i: (0, 0)
    return pl.pallas_call(
        _mlp_body,
        grid=(GRID_M,),
        in_specs=[
            pl.BlockSpec((1, D), rep),
            pl.BlockSpec((BM, D), lambda i: (i, 0)),
            pl.BlockSpec((BM, 1), lambda i: (i, 0)),
            pl.BlockSpec((D, H), rep),
            pl.BlockSpec((D, H), rep),
            pl.BlockSpec((1, H), rep),
            pl.BlockSpec((H, 1), rep),
            pl.BlockSpec((1, 1), rep),
            pl.BlockSpec((D, H), rep),
            pl.BlockSpec((D, H), rep),
            pl.BlockSpec((1, H), rep),
            pl.BlockSpec((H, K), rep),
            pl.BlockSpec((1, K), rep),
        ],
        out_specs=[
            pl.BlockSpec((BM, K), lambda i: (i, 0)),
            pl.BlockSpec((BM, K), lambda i: (i, 0)),
        ],
        out_shape=[
            jax.ShapeDtypeStruct((N_ACC, K), jnp.float32),
            jax.ShapeDtypeStruct((N_ACC, K), jnp.float32),
        ],
    )(q2, emb, labf, wp1a, wp1b, bp1, wp2, bp2, wc1a, wc1b, bc1, wc2, bc2)


def _spmm_body(cols_hbm, rows_hbm, y_hbm, zeros_hbm, out_hbm,
               colv, rowv, gbuf, acc, yv,
               semI0, semI1, semI2, semI3, semI4, semI5, semI6, semI7,
               semG0, semG1, semG2, semG3, semG4, semG5, semG6, semG7,
               semS0, semS1, semS2, semS3, semS4, semS5, semS6, semS7):
    c = lax.axis_index("c")
    s = lax.axis_index("s")
    wid = c * 16 + s
    row0 = s * ROWS_PER_TILE
    semI = (semI0, semI1, semI2, semI3, semI4, semI5, semI6, semI7)
    semG = (semG0, semG1, semG2, semG3, semG4, semG5, semG6, semG7)
    semS = (semS0, semS1, semS2, semS3, semS4, semS5, semS6, semS7)
    pltpu.sync_copy(zeros_hbm, acc.at[pl.ds(row0, ROWS_PER_TILE)])
    pltpu.sync_copy(y_hbm.at[pl.ds(row0, ROWS_PER_TILE)],
                    yv.at[pl.ds(row0, ROWS_PER_TILE)])
    plsc.subcore_barrier()

    def stage_f(m, b):
        mbase = (wid * MPW + m) * MROWS
        pltpu.async_copy(cols_hbm.at[pl.ds(mbase, MROWS)], colv.at[b], semI[b])
        pltpu.async_copy(rows_hbm.at[pl.ds(mbase, MROWS)], rowv.at[b], semI[b])

    def stage_d(m, b):
        mbase = (wid * MPW + m) * MROWS
        pltpu.make_async_copy(cols_hbm.at[pl.ds(mbase, MROWS)], colv.at[b], semI[b]).wait()
        pltpu.make_async_copy(rows_hbm.at[pl.ds(mbase, MROWS)], rowv.at[b], semI[b]).wait()

    def fire_g(b):
        for j in range(MROWS):
            pltpu.async_copy(yv.at[colv.at[b, j]], gbuf.at[b, j], semG[b])

    def drain_g(b):
        for j in range(MROWS):
            pltpu.make_async_copy(yv.at[colv.at[b, j]], gbuf.at[b, j], semG[b]).wait()

    def fire_s(b):
        for j in range(MROWS):
            pltpu.async_copy(gbuf.at[b, j], acc.at[rowv.at[b, j]], semS[b], add=True)

    def drain_s(b):
        for j in range(MROWS):
            pltpu.make_async_copy(gbuf.at[b, j], acc.at[rowv.at[b, j]], semS[b]).wait()

    for b in range(RING):
        stage_f(b, b)

    def body(i, carry):
        m = i * RING
        for b in range(RING):
            stage_d(m + b, b)
            fire_g(b)
        for b in range(RING):
            drain_g(b)
            fire_s(b)
        for b in range(RING):
            @pl.when(m + RING + b < MPW)
            def _():
                drain_s(b)
                stage_f(m + RING + b, b)
        return carry

    lax.fori_loop(0, MPW // RING, body, 0)
    for b in range(RING):
        drain_s(b)
    plsc.subcore_barrier()
    pltpu.sync_copy(acc.at[pl.ds(row0, ROWS_PER_TILE)],
                    out_hbm.at[c, pl.ds(row0, ROWS_PER_TILE)])


_spmm = functools.partial(
    pl.kernel,
    mesh=plsc.VectorSubcoreMesh(core_axis_name="c", subcore_axis_name="s"),
    compiler_params=pltpu.CompilerParams(use_tc_tiling_on_sc=False),
    out_type=jax.ShapeDtypeStruct((2, N_ACC, K), jnp.float32),
    scratch_types=[
        pltpu.VMEM((RING, MROWS, CHUNK), jnp.int32),
        pltpu.VMEM((RING, MROWS, CHUNK), jnp.int32),
        pltpu.VMEM((RING, MROWS, CHUNK, K), jnp.float32),
        pltpu.VMEM_SHARED((N_ACC, K), jnp.float32),
        pltpu.VMEM_SHARED((N_ACC, K), jnp.float32),
    ] + [pltpu.SemaphoreType.DMA] * 24,
)(_spmm_body)


def _combine_body(p, amps, y_o):
    pv = p[...]
    y_o[...] = (pv[0] + pv[1]) * amps[...]


def _combine(p, amps):
    return pl.pallas_call(
        _combine_body,
        grid=(GRID_M,),
        in_specs=[
            pl.BlockSpec((2, BM, K), lambda i: (0, i, 0)),
            pl.BlockSpec((BM, K), lambda i: (i, 0)),
        ],
        out_specs=pl.BlockSpec((BM, K), lambda i: (i, 0)),
        out_shape=jax.ShapeDtypeStruct((N_ACC, K), jnp.float32),
    )(p, amps)


def _final_body(p, ra_o, ss_o):
    pv = p[...]
    u = pv[0] + pv[1]
    rowi = lax.broadcasted_iota(jnp.int32, (BM, 1), 0) + pl.program_id(0) * BM
    u = jnp.where(rowi < N, u, 0.0)
    ra_o[...] = jnp.sum(jnp.abs(u), axis=1, keepdims=True)
    blk = jnp.sum(u * u)
    prev = jnp.where(pl.program_id(0) == 0, 0.0, ss_o[0, 0])
    ss_o[0, 0] = prev + blk


def _final(p):
    return pl.pallas_call(
        _final_body,
        grid=(GRID_M,),
        in_specs=[pl.BlockSpec((2, BM, K), lambda i: (0, i, 0))],
        out_specs=[
            pl.BlockSpec((BM, 1), lambda i: (i, 0)),
            pl.BlockSpec(memory_space=pltpu.SMEM),
        ],
        out_shape=[
            jax.ShapeDtypeStruct((N_ACC, 1), jnp.float32),
            jax.ShapeDtypeStruct((1, 1), jnp.float32),
        ],
    )(p)


def _scale_body(ra, ss, o):
    ssv = ss[0, 0]
    inv = jnp.where(ssv > 0.0, lax.rsqrt(ssv), 1.0)
    o[...] = ra[...] * inv


def _scale(ra, ss):
    return pl.pallas_call(
        _scale_body,
        grid=(GRID_M,),
        in_specs=[
            pl.BlockSpec((BM, 1), lambda i: (i, 0)),
            pl.BlockSpec(memory_space=pltpu.SMEM),
        ],
        out_specs=pl.BlockSpec((BM, 1), lambda i: (i, 0)),
        out_shape=jax.ShapeDtypeStruct((N_ACC, 1), jnp.float32),
    )(ra, ss)


def kernel(q_emb, emb, edge_index, labels, Wc1, bc1, Wc2, bc2, Wp1, bp1, Wp2, bp2):
    q2 = q_emb.reshape(1, D)
    labf = labels.astype(jnp.float32).reshape(N, 1)
    amps, y = _mlp(
        q2, emb, labf,
        Wp1[:D], Wp1[D:], bp1.reshape(1, H), Wp2, bp2.reshape(1, 1),
        Wc1[:D], Wc1[D:], bc1.reshape(1, H), Wc2, bc2.reshape(1, K),
    )
    npad = E_PAD - E
    cols = jnp.concatenate([edge_index[1], jnp.zeros((npad,), jnp.int32)])
    rows = jnp.concatenate([edge_index[0], jnp.full((npad,), N, jnp.int32)])
    cols2 = cols.reshape(E_PAD // CHUNK, CHUNK)
    rows2 = rows.reshape(E_PAD // CHUNK, CHUNK)
    zeros = jnp.zeros((ROWS_PER_TILE, K), jnp.float32)
    for step in range(WALK_STEPS):
        p = _spmm(cols2, rows2, y, zeros)
        if step < WALK_STEPS - 1:
            y = _combine(p, amps)
    ra, ss = _final(p)
    out = _scale(ra, ss)
    return out[:N].reshape(N)
